# baseline (speedup 1.0000x reference)
import functools

import jax
import jax.numpy as jnp
from jax import lax
from jax.experimental import pallas as pl

B = 4
N = 2048
K = 1024
TILE = 256


def _rank_kernel(conf_ref, logits_t_ref, perm_ref, fidx_ref, preds_t_ref):
    b = pl.program_id(0)
    conf_row = conf_ref[0]
    iota_j = lax.broadcasted_iota(jnp.int32, (1, N), 1)
    l0 = logits_t_ref[0, 0:1, :]
    l1 = logits_t_ref[0, 1:2, :]

    ranks = []
    for t in range(N // TILE):
        ci = conf_row[0:1, t * TILE:(t + 1) * TILE].reshape(TILE, 1)
        ii = (lax.broadcasted_iota(jnp.int32, (TILE, 1), 0) + t * TILE)
        gt = (conf_row > ci).astype(jnp.int32)
        eqb = ((conf_row == ci) & (iota_j < ii)).astype(jnp.int32)
        ranks.append(jnp.sum(gt + eqb, axis=1))
    rank_row = jnp.concatenate(ranks).reshape(1, N)

    for t in range(N // TILE):
        rr = (lax.broadcasted_iota(jnp.int32, (TILE, 1), 0) + t * TILE)
        eq = (rank_row == rr)
        eqf = eq.astype(jnp.float32)
        perm_t = jnp.sum(jnp.where(eq, iota_j, 0), axis=1)
        p0 = jnp.sum(eqf * l0, axis=1)
        p1 = jnp.sum(eqf * l1, axis=1)
        perm_ref[0, 0, pl.ds(t * TILE, TILE)] = perm_t
        preds_t_ref[0, 0, pl.ds(t * TILE, TILE)] = p0
        preds_t_ref[0, 1, pl.ds(t * TILE, TILE)] = p1
        if t < K // TILE:
            fidx_ref[0, 0, pl.ds(t * TILE, TILE)] = perm_t + b * N


def _gather_kernel(perm_ref, x_ref, out_ref):
    iota_j = lax.broadcasted_iota(jnp.int32, (1, N), 1)
    for t in range(K // TILE):
        sel = perm_ref[0, 0, pl.ds(t * TILE, TILE)].reshape(TILE, 1)
        onehot = (sel == iota_j).astype(jnp.float32)
        out_ref[0, pl.ds(t * TILE, TILE), :] = lax.dot_general(
            onehot, x_ref[0],
            dimension_numbers=(((1,), (0,)), ((), ())),
            preferred_element_type=jnp.float32)


def kernel(x_feat, logits_feat):
    probs = jax.nn.softmax(logits_feat, axis=-1)
    conf = jnp.max(probs, axis=-1).reshape(B, 1, N)
    logits_t = logits_feat.transpose(0, 2, 1)

    perm, fidx, preds_t = pl.pallas_call(
        _rank_kernel,
        grid=(B,),
        in_specs=[
            pl.BlockSpec((1, 1, N), lambda b: (b, 0, 0)),
            pl.BlockSpec((1, 2, N), lambda b: (b, 0, 0)),
        ],
        out_specs=[
            pl.BlockSpec((1, 1, N), lambda b: (b, 0, 0)),
            pl.BlockSpec((1, 1, K), lambda b: (b, 0, 0)),
            pl.BlockSpec((1, 2, N), lambda b: (b, 0, 0)),
        ],
        out_shape=[
            jax.ShapeDtypeStruct((B, 1, N), jnp.int32),
            jax.ShapeDtypeStruct((B, 1, K), jnp.int32),
            jax.ShapeDtypeStruct((B, 2, N), jnp.float32),
        ],
    )(conf, logits_t)

    D = x_feat.shape[-1]
    DT = 512
    sf = pl.pallas_call(
        _gather_kernel,
        grid=(B, D // DT),
        in_specs=[
            pl.BlockSpec((1, 1, K), lambda b, d: (b, 0, 0)),
            pl.BlockSpec((1, N, DT), lambda b, d: (b, 0, d)),
        ],
        out_specs=pl.BlockSpec((1, K, DT), lambda b, d: (b, 0, d)),
        out_shape=jax.ShapeDtypeStruct((B, K, D), jnp.float32),
    )(perm, x_feat)

    preds = preds_t.transpose(0, 2, 1)
    preds_1 = preds[:, :K, :]
    preds_0 = preds[:, K:, :]
    return sf, preds_1, preds_0


# speedup vs baseline: 1.0811x; 1.0811x over previous
import functools

import jax
import jax.numpy as jnp
from jax import lax
from jax.experimental import pallas as pl
from jax.experimental.pallas import tpu as pltpu
from jax.experimental.pallas import tpu_sc as plsc

B = 4
N = 2048
K = 1024
TILE = 256


def _rank_kernel(conf_ref, logits_t_ref, perm_ref, fidx_ref, preds_t_ref):
    b = pl.program_id(0)
    conf_row = conf_ref[0]
    iota_j = lax.broadcasted_iota(jnp.int32, (1, N), 1)
    l0 = logits_t_ref[0, 0:1, :]
    l1 = logits_t_ref[0, 1:2, :]

    ranks = []
    for t in range(N // TILE):
        ci = conf_row[0:1, t * TILE:(t + 1) * TILE].reshape(TILE, 1)
        ii = (lax.broadcasted_iota(jnp.int32, (TILE, 1), 0) + t * TILE)
        gt = (conf_row > ci).astype(jnp.int32)
        eqb = ((conf_row == ci) & (iota_j < ii)).astype(jnp.int32)
        ranks.append(jnp.sum(gt + eqb, axis=1))
    rank_row = jnp.concatenate(ranks).reshape(1, N)

    for t in range(N // TILE):
        rr = (lax.broadcasted_iota(jnp.int32, (TILE, 1), 0) + t * TILE)
        eq = (rank_row == rr)
        eqf = eq.astype(jnp.float32)
        perm_t = jnp.sum(jnp.where(eq, iota_j, 0), axis=1)
        p0 = jnp.sum(eqf * l0, axis=1)
        p1 = jnp.sum(eqf * l1, axis=1)
        perm_ref[0, 0, pl.ds(t * TILE, TILE)] = perm_t
        preds_t_ref[0, 0, pl.ds(t * TILE, TILE)] = p0
        preds_t_ref[0, 1, pl.ds(t * TILE, TILE)] = p1
        if t < K // TILE:
            fidx_ref[0, 0, pl.ds(t * TILE, TILE)] = perm_t + b * N


_NC = 2
_NS = 16
_NW = _NC * _NS
_RPW = (B * K) // _NW
_CH = 16
_NCH = _RPW // _CH


def _sc_gather_body(x_hbm, idx_hbm, out_hbm, idx_v, buf_v, sem_a, sem_b):
    wid = lax.axis_index("s") * _NC + lax.axis_index("c")
    base = wid * _RPW
    pltpu.sync_copy(idx_hbm.at[pl.ds(base, _RPW)], idx_v)
    sems = (sem_a, sem_b)
    prev = None
    for ch in range(_NCH):
        dma = pltpu.async_copy(
            x_hbm.at[idx_v.at[pl.ds(ch * _CH, _CH)]],
            buf_v.at[ch % 2], sems[ch % 2])
        if prev is not None:
            pch, pdma = prev
            pdma.wait()
            pltpu.sync_copy(buf_v.at[pch % 2],
                            out_hbm.at[pl.ds(base + pch * _CH, _CH)])
        prev = (ch, dma)
    pch, pdma = prev
    pdma.wait()
    pltpu.sync_copy(buf_v.at[pch % 2],
                    out_hbm.at[pl.ds(base + pch * _CH, _CH)])


def kernel(x_feat, logits_feat):
    probs = jax.nn.softmax(logits_feat, axis=-1)
    conf = jnp.max(probs, axis=-1).reshape(B, 1, N)
    logits_t = logits_feat.transpose(0, 2, 1)

    perm, fidx, preds_t = pl.pallas_call(
        _rank_kernel,
        grid=(B,),
        in_specs=[
            pl.BlockSpec((1, 1, N), lambda b: (b, 0, 0)),
            pl.BlockSpec((1, 2, N), lambda b: (b, 0, 0)),
        ],
        out_specs=[
            pl.BlockSpec((1, 1, N), lambda b: (b, 0, 0)),
            pl.BlockSpec((1, 1, K), lambda b: (b, 0, 0)),
            pl.BlockSpec((1, 2, N), lambda b: (b, 0, 0)),
        ],
        out_shape=[
            jax.ShapeDtypeStruct((B, 1, N), jnp.int32),
            jax.ShapeDtypeStruct((B, 1, K), jnp.int32),
            jax.ShapeDtypeStruct((B, 2, N), jnp.float32),
        ],
    )(conf, logits_t)

    D = x_feat.shape[-1]
    gather_call = functools.partial(
        pl.kernel,
        mesh=plsc.VectorSubcoreMesh(core_axis_name="c", subcore_axis_name="s"),
        out_type=jax.ShapeDtypeStruct((B * K, D), jnp.float32),
        scratch_types=[
            pltpu.VMEM((_RPW,), jnp.int32),
            pltpu.VMEM((2, _CH, D), jnp.float32),
            pltpu.SemaphoreType.DMA,
            pltpu.SemaphoreType.DMA,
        ],
    )(_sc_gather_body)
    sf = gather_call(x_feat.reshape(B * N, D),
                     fidx.reshape(B * K)).reshape(B, K, D)

    preds = preds_t.transpose(0, 2, 1)
    preds_1 = preds[:, :K, :]
    preds_0 = preds[:, K:, :]
    return sf, preds_1, preds_0


# speedup vs baseline: 1.1377x; 1.0524x over previous
import functools

import jax
import jax.numpy as jnp
from jax import lax
from jax.experimental import pallas as pl
from jax.experimental.pallas import tpu as pltpu
from jax.experimental.pallas import tpu_sc as plsc

B = 4
N = 2048
K = 1024
TILE = 256


def _rank_kernel(conf_row_ref, conf_col_ref, hi_ref, lo_ref,
                 preds_ref, fidx_ref):
    b = pl.program_id(0)
    conf_row = conf_row_ref[0]
    iota_row = lax.broadcasted_iota(jnp.int32, (1, N), 1)

    ones_row = jnp.ones((1, TILE), jnp.bfloat16)
    rankf = jnp.zeros((1, N), jnp.float32)
    for t in range(N // TILE):
        cj = conf_col_ref[0, t * TILE:(t + 1) * TILE, 0:1]
        jj = lax.broadcasted_iota(jnp.int32, (TILE, 1), 0) + t * TILE
        gt = cj > conf_row
        eqb = (cj == conf_row) & (jj < iota_row)
        maskh = (gt | eqb).astype(jnp.bfloat16)
        rankf = rankf + lax.dot_general(
            ones_row, maskh, (((1,), (0,)), ((), ())),
            preferred_element_type=jnp.float32)

    hi_mat = hi_ref[0]
    lo_mat = lo_ref[0]
    for t in range(N // TILE):
        rr = (lax.broadcasted_iota(jnp.int32, (TILE, 1), 0)
              + t * TILE).astype(jnp.float32)
        eqh = (rankf == rr).astype(jnp.bfloat16)
        out3 = (lax.dot_general(eqh, hi_mat, (((1,), (0,)), ((), ())),
                                preferred_element_type=jnp.float32)
                + lax.dot_general(eqh, lo_mat, (((1,), (0,)), ((), ())),
                                  preferred_element_type=jnp.float32))
        preds_ref[0, t * TILE:(t + 1) * TILE, :] = out3[:, 0:2]
        if t < K // TILE:
            fidx_ref[0, t * TILE:(t + 1) * TILE, 0:1] = (
                jnp.round(out3[:, 2:3]).astype(jnp.int32) + b * N)


_NC = 2
_NS = 16
_NW = _NC * _NS
_RPW = (B * K) // _NW
_CH = 16
_NCH = _RPW // _CH


def _sc_gather_body(x_hbm, idx_hbm, out_hbm, idx_v, buf_v, sem_a, sem_b):
    wid = lax.axis_index("s") * _NC + lax.axis_index("c")
    base = wid * _RPW
    pltpu.sync_copy(idx_hbm.at[pl.ds(base, _RPW)], idx_v)
    sems = (sem_a, sem_b)
    prev = None
    for ch in range(_NCH):
        dma = pltpu.async_copy(
            x_hbm.at[idx_v.at[pl.ds(ch * _CH, _CH)]],
            buf_v.at[ch % 2], sems[ch % 2])
        if prev is not None:
            pch, pdma = prev
            pdma.wait()
            pltpu.sync_copy(buf_v.at[pch % 2],
                            out_hbm.at[pl.ds(base + pch * _CH, _CH)])
        prev = (ch, dma)
    pch, pdma = prev
    pdma.wait()
    pltpu.sync_copy(buf_v.at[pch % 2],
                    out_hbm.at[pl.ds(base + pch * _CH, _CH)])


def kernel(x_feat, logits_feat):
    probs = jax.nn.softmax(logits_feat, axis=-1)
    conf = jnp.max(probs, axis=-1)
    conf_row = conf.reshape(B, 1, N)
    conf_col = conf.reshape(B, N, 1)
    iota_col = lax.broadcasted_iota(jnp.float32, (B, N, 1), 1)
    aug = jnp.concatenate([logits_feat, iota_col], axis=2)
    aug_hi = aug.astype(jnp.bfloat16)
    aug_lo = (aug - aug_hi.astype(jnp.float32)).astype(jnp.bfloat16)

    preds, fidx = pl.pallas_call(
        _rank_kernel,
        grid=(B,),
        in_specs=[
            pl.BlockSpec((1, 1, N), lambda b: (b, 0, 0)),
            pl.BlockSpec((1, N, 1), lambda b: (b, 0, 0)),
            pl.BlockSpec((1, N, 3), lambda b: (b, 0, 0)),
            pl.BlockSpec((1, N, 3), lambda b: (b, 0, 0)),
        ],
        out_specs=[
            pl.BlockSpec((1, N, 2), lambda b: (b, 0, 0)),
            pl.BlockSpec((1, K, 1), lambda b: (b, 0, 0)),
        ],
        out_shape=[
            jax.ShapeDtypeStruct((B, N, 2), jnp.float32),
            jax.ShapeDtypeStruct((B, K, 1), jnp.int32),
        ],
    )(conf_row, conf_col, aug_hi, aug_lo)

    D = x_feat.shape[-1]
    gather_call = functools.partial(
        pl.kernel,
        mesh=plsc.VectorSubcoreMesh(core_axis_name="c", subcore_axis_name="s"),
        out_type=jax.ShapeDtypeStruct((B * K, D), jnp.float32),
        scratch_types=[
            pltpu.VMEM((_RPW,), jnp.int32),
            pltpu.VMEM((2, _CH, D), jnp.float32),
            pltpu.SemaphoreType.DMA,
            pltpu.SemaphoreType.DMA,
        ],
    )(_sc_gather_body)
    sf = gather_call(x_feat.reshape(B * N, D),
                     fidx.reshape(B * K)).reshape(B, K, D)

    preds_1 = preds[:, :K, :]
    preds_0 = preds[:, K:, :]
    return sf, preds_1, preds_0


# speedup vs baseline: 1.2921x; 1.1357x over previous
import functools

import jax
import jax.numpy as jnp
from jax import lax
from jax.experimental import pallas as pl
from jax.experimental.pallas import tpu as pltpu
from jax.experimental.pallas import tpu_sc as plsc

B = 4
N = 2048
K = 1024
TILE = 256


def _rank_kernel(conf_row_ref, conf_col_ref, lg_ref, preds_ref, idx_ref):
    b = pl.program_id(0)
    conf_row = conf_row_ref[0]
    iota_row = lax.broadcasted_iota(jnp.int32, (1, N), 1)

    ones_row = jnp.ones((1, TILE), jnp.bfloat16)
    rankf = jnp.zeros((1, N), jnp.float32)
    for t in range(N // TILE):
        cj = conf_col_ref[0, t * TILE:(t + 1) * TILE, 0:1]
        jj = lax.broadcasted_iota(jnp.int32, (TILE, 1), 0) + t * TILE
        gt = cj > conf_row
        eqb = (cj == conf_row) & (jj < iota_row)
        maskh = (gt | eqb).astype(jnp.bfloat16)
        rankf = rankf + lax.dot_general(
            ones_row, maskh, (((1,), (0,)), ((), ())),
            preferred_element_type=jnp.float32)

    lg = lg_ref[0]
    lg_hi = lg.astype(jnp.bfloat16)
    r1 = lg - lg_hi.astype(jnp.float32)
    lg_mid = r1.astype(jnp.bfloat16)
    lg_lo = (r1 - lg_mid.astype(jnp.float32)).astype(jnp.bfloat16)
    ii = lax.broadcasted_iota(jnp.int32, (N, 1), 0)
    ia = (ii // 256).astype(jnp.bfloat16)
    ic = (ii % 256).astype(jnp.bfloat16)
    vals = jnp.concatenate(
        [lg_hi, lg_mid, lg_lo, ia, ic], axis=1)

    for t in range(N // TILE):
        rr = (lax.broadcasted_iota(jnp.int32, (TILE, 1), 0)
              + t * TILE).astype(jnp.float32)
        oh = (rankf == rr).astype(jnp.bfloat16)
        out = lax.dot_general(
            oh, vals, (((1,), (0,)), ((), ())),
            preferred_element_type=jnp.float32)
        preds_ref[0, t * TILE:(t + 1) * TILE, :] = (
            out[:, 0:2] + out[:, 2:4] + out[:, 4:6])
        idx_ref[0, t * TILE:(t + 1) * TILE, :] = (
            256 * out[:, 6:7] + out[:, 7:8]).astype(jnp.int32) + b * N


_NC = 2
_NS = 16
_NW = _NC * _NS
_FPW = K // 8
_CH = 16
_NCH = _FPW // _CH


def _sc_body(x_hbm, idx_hbm, sf_hbm, fidx_v, buf_v, sem_a, sem_b):
    cid = lax.axis_index("c")
    sid = lax.axis_index("s")
    wid = sid * _NC + cid

    fbase = (wid // 8) * N + (wid % 8) * _FPW
    obase = wid * _FPW
    pltpu.sync_copy(idx_hbm.at[pl.ds(fbase, _FPW)], fidx_v)
    sems = (sem_a, sem_b)
    prev = None
    for ch in range(_NCH):
        dma = pltpu.async_copy(
            x_hbm.at[fidx_v.at[pl.ds(ch * _CH, _CH)]],
            buf_v.at[ch % 2], sems[ch % 2])
        if prev is not None:
            pch, pdma = prev
            pdma.wait()
            pltpu.sync_copy(buf_v.at[pch % 2],
                            sf_hbm.at[pl.ds(obase + pch * _CH, _CH)])
        prev = (ch, dma)
    pch, pdma = prev
    pdma.wait()
    pltpu.sync_copy(buf_v.at[pch % 2],
                    sf_hbm.at[pl.ds(obase + pch * _CH, _CH)])


def kernel(x_feat, logits_feat):
    probs = jax.nn.softmax(logits_feat, axis=-1)
    conf = jnp.max(probs, axis=-1)
    conf_row = conf.reshape(B, 1, N)
    conf_col = conf.reshape(B, N, 1)

    preds, idx = pl.pallas_call(
        _rank_kernel,
        grid=(B,),
        in_specs=[
            pl.BlockSpec((1, 1, N), lambda b: (b, 0, 0)),
            pl.BlockSpec((1, N, 1), lambda b: (b, 0, 0)),
            pl.BlockSpec((1, N, 2), lambda b: (b, 0, 0)),
        ],
        out_specs=[
            pl.BlockSpec((1, N, 2), lambda b: (b, 0, 0)),
            pl.BlockSpec((1, N, 1), lambda b: (b, 0, 0)),
        ],
        out_shape=[
            jax.ShapeDtypeStruct((B, N, 2), jnp.float32),
            jax.ShapeDtypeStruct((B, N, 1), jnp.int32),
        ],
    )(conf_row, conf_col, logits_feat)

    D = x_feat.shape[-1]
    gather_call = functools.partial(
        pl.kernel,
        mesh=plsc.VectorSubcoreMesh(core_axis_name="c", subcore_axis_name="s"),
        out_type=[
            jax.ShapeDtypeStruct((B * K, D), jnp.float32),
        ],
        scratch_types=[
            pltpu.VMEM((_FPW,), jnp.int32),
            pltpu.VMEM((2, _CH, D), jnp.float32),
            pltpu.SemaphoreType.DMA,
            pltpu.SemaphoreType.DMA,
        ],
    )(_sc_body)
    (sf,) = gather_call(x_feat.reshape(B * N, D), idx.reshape(B * N))
    sf = sf.reshape(B, K, D)
    return sf, preds[:, :K, :], preds[:, K:, :]


# speedup vs baseline: 1.2925x; 1.0002x over previous
import functools

import jax
import jax.numpy as jnp
from jax import lax
from jax.experimental import pallas as pl
from jax.experimental.pallas import tpu as pltpu
from jax.experimental.pallas import tpu_sc as plsc

B = 4
N = 2048
K = 1024
TILE = 256


def _rank_kernel(conf_row_ref, conf_col_ref, lg_ref, preds_ref, idx_ref):
    b = pl.program_id(0)
    conf_row = conf_row_ref[0]
    iota_row = lax.broadcasted_iota(jnp.int32, (1, N), 1)

    ones_row = jnp.ones((1, TILE), jnp.bfloat16)
    rankf = jnp.zeros((1, N), jnp.float32)
    for t in range(N // TILE):
        cj = conf_col_ref[0, t * TILE:(t + 1) * TILE, 0:1]
        jj = lax.broadcasted_iota(jnp.int32, (TILE, 1), 0) + t * TILE
        gt = cj > conf_row
        eqb = (cj == conf_row) & (jj < iota_row)
        maskh = (gt | eqb).astype(jnp.bfloat16)
        rankf = rankf + lax.dot_general(
            ones_row, maskh, (((1,), (0,)), ((), ())),
            preferred_element_type=jnp.float32)

    lg = lg_ref[0]
    lg_hi = lg.astype(jnp.bfloat16)
    r1 = lg - lg_hi.astype(jnp.float32)
    lg_mid = r1.astype(jnp.bfloat16)
    lg_lo = (r1 - lg_mid.astype(jnp.float32)).astype(jnp.bfloat16)
    ii = lax.broadcasted_iota(jnp.int32, (N, 1), 0)
    ia = (ii // 256).astype(jnp.bfloat16)
    ic = (ii % 256).astype(jnp.bfloat16)
    vals = jnp.concatenate(
        [lg_hi, lg_mid, lg_lo, ia, ic], axis=1)

    for t in range(N // TILE):
        rr = (lax.broadcasted_iota(jnp.int32, (TILE, 1), 0)
              + t * TILE).astype(jnp.float32)
        oh = (rankf == rr).astype(jnp.bfloat16)
        out = lax.dot_general(
            oh, vals, (((1,), (0,)), ((), ())),
            preferred_element_type=jnp.float32)
        preds_ref[0, t * TILE:(t + 1) * TILE, :] = (
            out[:, 0:2] + out[:, 2:4] + out[:, 4:6])
        idx_ref[0, t * TILE:(t + 1) * TILE, :] = (
            256 * out[:, 6:7] + out[:, 7:8]).astype(jnp.int32) + b * N


_NC = 2
_NS = 16
_NW = _NC * _NS
_FPW = K // 8
_CH = 16
_NCH = _FPW // _CH


def _sc_body(x_hbm, idx_hbm, sf_hbm, fidx_v, buf_v,
             gsem_a, gsem_b, osem_a, osem_b):
    cid = lax.axis_index("c")
    sid = lax.axis_index("s")
    wid = sid * _NC + cid

    fbase = (wid // 8) * N + (wid % 8) * _FPW
    obase = wid * _FPW
    pltpu.sync_copy(idx_hbm.at[pl.ds(fbase, _FPW)], fidx_v)
    gsems = (gsem_a, gsem_b)
    osems = (osem_a, osem_b)
    gd = [None, None]
    od = [None, None]
    for ch in range(_NCH):
        s = ch % 2
        if od[s] is not None:
            od[s].wait()
        gd[s] = pltpu.async_copy(
            x_hbm.at[fidx_v.at[pl.ds(ch * _CH, _CH)]],
            buf_v.at[s], gsems[s])
        if ch > 0:
            p = 1 - s
            gd[p].wait()
            od[p] = pltpu.async_copy(
                buf_v.at[p],
                sf_hbm.at[pl.ds(obase + (ch - 1) * _CH, _CH)], osems[p])
    s = (_NCH - 1) % 2
    gd[s].wait()
    od[s] = pltpu.async_copy(
        buf_v.at[s], sf_hbm.at[pl.ds(obase + (_NCH - 1) * _CH, _CH)],
        osems[s])
    od[0].wait()
    od[1].wait()


def kernel(x_feat, logits_feat):
    probs = jax.nn.softmax(logits_feat, axis=-1)
    conf = jnp.max(probs, axis=-1)
    conf_row = conf.reshape(B, 1, N)
    conf_col = conf.reshape(B, N, 1)

    preds, idx = pl.pallas_call(
        _rank_kernel,
        grid=(B,),
        in_specs=[
            pl.BlockSpec((1, 1, N), lambda b: (b, 0, 0)),
            pl.BlockSpec((1, N, 1), lambda b: (b, 0, 0)),
            pl.BlockSpec((1, N, 2), lambda b: (b, 0, 0)),
        ],
        out_specs=[
            pl.BlockSpec((1, N, 2), lambda b: (b, 0, 0)),
            pl.BlockSpec((1, N, 1), lambda b: (b, 0, 0)),
        ],
        out_shape=[
            jax.ShapeDtypeStruct((B, N, 2), jnp.float32),
            jax.ShapeDtypeStruct((B, N, 1), jnp.int32),
        ],
    )(conf_row, conf_col, logits_feat)

    D = x_feat.shape[-1]
    gather_call = functools.partial(
        pl.kernel,
        mesh=plsc.VectorSubcoreMesh(core_axis_name="c", subcore_axis_name="s"),
        out_type=[
            jax.ShapeDtypeStruct((B * K, D), jnp.float32),
        ],
        scratch_types=[
            pltpu.VMEM((_FPW,), jnp.int32),
            pltpu.VMEM((2, _CH, D), jnp.float32),
            pltpu.SemaphoreType.DMA,
            pltpu.SemaphoreType.DMA,
            pltpu.SemaphoreType.DMA,
            pltpu.SemaphoreType.DMA,
        ],
    )(_sc_body)
    (sf,) = gather_call(x_feat.reshape(B * N, D), idx.reshape(B * N))
    sf = sf.reshape(B, K, D)
    return sf, preds[:, :K, :], preds[:, K:, :]
